# baseline (speedup 1.0000x reference)
import jax
import jax.numpy as jnp
from jax import lax
from jax.experimental import pallas as pl
from jax.experimental.pallas import tpu as pltpu

C = 192
H = 224
W = 224
HEADS = 4
CH = C // HEADS
TR = 8
NT = H // TR
R = TR + 2
N_PIX = H * W
KS = (CH // 2, CH * 2 // 3, CH * 3 // 4, CH * 4 // 5)


def _chain_kernel(xp_ref, x_ref, xn_ref, lw_ref, lb_ref, w_ref, b_ref,
                  w9_ref, bd_ref, o_ref, s_ref):
    i = pl.program_id(0)
    y = jnp.concatenate([xp_ref[...].reshape(C, 1, W), x_ref[...],
                         xn_ref[...].reshape(C, 1, W)], axis=1)
    x = y.reshape(C, R * W)
    mu = jnp.mean(x, axis=0, keepdims=True)
    var = jnp.mean(x * x, axis=0, keepdims=True) - mu * mu
    x = (x - mu) * lax.rsqrt(var + 1e-6) * lw_ref[...] + lb_ref[...]
    z = lax.dot_general(w_ref[...], x, (((1,), (0,)), ((), ())),
                        preferred_element_type=jnp.float32) + b_ref[...]
    yc = z.reshape(C, R, W)
    row = lax.broadcasted_iota(jnp.int32, (1, R, 1), 1)
    topf = (i > 0).astype(jnp.float32)
    botf = (i < NT - 1).astype(jnp.float32)
    rowmask = jnp.where(row == 0, topf, 1.0) * \
        jnp.where(row == R - 1, botf, 1.0)
    yc = yc * rowmask
    lane = lax.broadcasted_iota(jnp.int32, (1, 1, W), 2)
    zz = None
    for dc in range(3):
        s = None
        for dr in range(3):
            t = w9_ref[:, 3 * dr + dc:3 * dr + dc + 1][:, :, None] \
                * yc[:, dr:dr + TR, :]
            s = t if s is None else s + t
        if dc == 0:
            s = jnp.where(lane > 0, pltpu.roll(s, 1, 2), 0.0)
        elif dc == 2:
            s = jnp.where(lane < W - 1, pltpu.roll(s, W - 1, 2), 0.0)
        zz = s if zz is None else zz + s
    o = zz + bd_ref[...][:, :, None]
    o_ref[...] = o
    of = o.reshape(C, TR * W)

    @pl.when(i == 0)
    def _():
        s_ref[...] = jnp.zeros_like(s_ref)

    s_ref[...] += jnp.sum(of * of, axis=1, keepdims=True)


def _g_kernel(q_ref, k_ref, g_ref):
    i = pl.program_id(0)
    q = q_ref[...].reshape(C, TR * W)
    k = k_ref[...].reshape(C, TR * W)

    @pl.when(i == 0)
    def _():
        g_ref[...] = jnp.zeros_like(g_ref)

    for h in range(HEADS):
        g_ref[h, :, :] += lax.dot_general(
            q[h * CH:(h + 1) * CH, :], k[h * CH:(h + 1) * CH, :],
            (((1,), (1,)), ((), ())), preferred_element_type=jnp.float32)


def _p2_kernel(g_ref, sq_ref, sk_ref, t_ref, aw_ref, wproj_ref, m_ref):
    rq = 1.0 / jnp.maximum(jnp.sqrt(sq_ref[...]), 1e-12)
    rk = 1.0 / jnp.maximum(jnp.sqrt(sk_ref[...]), 1e-12)
    temp = t_ref[0:1, 0:1]
    rows = []
    for h in range(HEADS):
        d = lax.dot_general(rq[h * CH:(h + 1) * CH, :],
                            rk[h * CH:(h + 1) * CH, :],
                            (((1,), (1,)), ((), ())),
                            preferred_element_type=jnp.float32)
        rows.append(g_ref[h, :, :] * d)
    attn = jnp.concatenate(rows, axis=0) * temp

    a1 = attn[:, :, None]
    a2 = attn[:, None, :]
    ei = lax.broadcasted_iota(jnp.int32, (1, CH, CH), 1)
    di = lax.broadcasted_iota(jnp.int32, (1, CH, CH), 2)
    first = (a1 > a2) | ((a1 == a2) & (ei < di))
    rank = jnp.sum(first.astype(jnp.float32), axis=1)

    m = jnp.max(attn, axis=1, keepdims=True)
    e = jnp.exp(attn - m)
    wsum = jnp.zeros_like(attn)
    for idx, kk in enumerate(KS):
        mask = (rank < kk).astype(jnp.float32)
        den = jnp.sum(e * mask, axis=1, keepdims=True)
        wsum += aw_ref[idx:idx + 1, 0:1] * (mask / den)
    a = e * wsum

    for h in range(HEADS):
        mh = lax.dot_general(wproj_ref[:, h * CH:(h + 1) * CH],
                             a[h * CH:(h + 1) * CH, :],
                             (((1,), (0,)), ((), ())),
                             preferred_element_type=jnp.float32)
        m_ref[:, h * CH:(h + 1) * CH] = mh


def _p3_kernel(v_ref, xc_ref, xd_ref, m_ref, bp_ref, w1_ref, w2_ref,
               f1_ref, s_ref, acc_ref):
    i = pl.program_id(0)
    vf = v_ref[...].reshape(C, TR * W)
    f1 = lax.dot_general(m_ref[...], vf, (((1,), (0,)), ((), ())),
                         preferred_element_type=jnp.float32) + bp_ref[...]
    f1 = f1.reshape(C, TR, W) + xc_ref[...]
    f1_ref[...] = f1

    @pl.when(i == 0)
    def _():
        acc_ref[...] = jnp.zeros_like(acc_ref)

    acc_ref[...] += (jnp.sum(f1.reshape(C, TR * W), axis=1, keepdims=True)
                     + jnp.sum(xd_ref[...].reshape(C, TR * W), axis=1,
                               keepdims=True))

    @pl.when(i == NT - 1)
    def _():
        pooled = acc_ref[...] * (1.0 / N_PIX)
        hid = jnp.maximum(
            lax.dot_general(w1_ref[...], pooled, (((1,), (0,)), ((), ())),
                            preferred_element_type=jnp.float32), 0.0)
        att = lax.dot_general(w2_ref[...], hid, (((1,), (0,)), ((), ())),
                              preferred_element_type=jnp.float32)
        pair = jnp.concatenate([att[0:C, :], att[C:2 * C, :]], axis=1)
        mx = jnp.max(pair, axis=1, keepdims=True)
        ee = jnp.exp(pair - mx)
        s_ref[...] = ee / jnp.sum(ee, axis=1, keepdims=True)


def _p5_kernel(f1_ref, xd_ref, s_ref, o_ref):
    s0 = s_ref[:, 0:1][:, :, None]
    s1 = s_ref[:, 1:2][:, :, None]
    o_ref[...] = f1_ref[...] * s0 + xd_ref[...] * s1


def kernel(T_E_input, C_E_input, T_D_input, t_norm_w, t_norm_b, c_norm_w,
           c_norm_b, Wq, bq, Wq_dw, bq_dw, Wk, bk, Wk_dw, bk_dw, Wv, bv,
           Wv_dw, bv_dw, temperature, Wproj, bproj, a1, a2, a3, a4,
           sk_w1, sk_w2):
    f32 = jnp.float32
    xt = T_E_input.reshape(C, H, W)
    xc = C_E_input.reshape(C, H, W)
    xd = T_D_input.reshape(C, H, W)
    col = lambda v: v.reshape(-1, 1).astype(f32)
    temp2 = temperature.reshape(1, 1)
    awts = jnp.concatenate([a1, a2, a3, a4]).reshape(4, 1)

    xt4 = xt.reshape(C, H, 1, W)
    xc4 = xc.reshape(C, H, 1, W)
    main = pl.BlockSpec((C, TR, W), lambda i: (0, i, 0))
    halo_t = pl.BlockSpec((C, 1, 1, W),
                          lambda i: (0, jnp.maximum(i * TR - 1, 0), 0, 0))
    halo_b = pl.BlockSpec((C, 1, 1, W),
                          lambda i: (0, jnp.minimum(i * TR + TR, H - 1), 0, 0))

    def const(shape):
        nd = len(shape)
        return pl.BlockSpec(shape, lambda i: (0,) * nd)

    seq = pltpu.CompilerParams(dimension_semantics=("arbitrary",),
                               vmem_limit_bytes=100 * 1024 * 1024)

    chain = pl.pallas_call(
        _chain_kernel,
        grid=(NT,),
        in_specs=[halo_t, main, halo_b,
                  const((C, 1)), const((C, 1)), const((C, C)), const((C, 1)),
                  const((C, 9)), const((C, 1))],
        out_specs=[main, const((C, 1))],
        out_shape=[jax.ShapeDtypeStruct((C, H, W), f32),
                   jax.ShapeDtypeStruct((C, 1), f32)],
        compiler_params=seq,
    )

    q, sq = chain(xt4, xt, xt4, col(t_norm_w), col(t_norm_b),
                  Wq, col(bq), Wq_dw.reshape(C, 9), col(bq_dw))
    k, sk = chain(xc4, xc, xc4, col(c_norm_w), col(c_norm_b),
                  Wk, col(bk), Wk_dw.reshape(C, 9), col(bk_dw))
    v, _ = chain(xc4, xc, xc4, col(c_norm_w), col(c_norm_b),
                 Wv, col(bv), Wv_dw.reshape(C, 9), col(bv_dw))

    g = pl.pallas_call(
        _g_kernel,
        grid=(NT,),
        in_specs=[main, main],
        out_specs=const((HEADS, CH, CH)),
        out_shape=jax.ShapeDtypeStruct((HEADS, CH, CH), f32),
        compiler_params=seq,
    )(q, k)

    m = pl.pallas_call(
        _p2_kernel,
        out_shape=jax.ShapeDtypeStruct((C, C), f32),
    )(g, sq, sk, temp2, awts, Wproj)

    f1, s = pl.pallas_call(
        _p3_kernel,
        grid=(NT,),
        in_specs=[main, main, main, const((C, C)), const((C, 1)),
                  const(sk_w1.shape), const(sk_w2.shape)],
        out_specs=[main, const((C, 2))],
        out_shape=[jax.ShapeDtypeStruct((C, H, W), f32),
                   jax.ShapeDtypeStruct((C, 2), f32)],
        scratch_shapes=[pltpu.VMEM((C, 1), f32)],
        compiler_params=seq,
    )(v, xc, xd, m, col(bproj), sk_w1, sk_w2)

    out = pl.pallas_call(
        _p5_kernel,
        grid=(NT,),
        in_specs=[main, main, const((C, 2))],
        out_specs=main,
        out_shape=jax.ShapeDtypeStruct((C, H, W), f32),
        compiler_params=seq,
    )(f1, xd, s)

    return out.reshape(1, C, H, W)


# speedup vs baseline: 1.0798x; 1.0798x over previous
import jax
import jax.numpy as jnp
from jax import lax
from jax.experimental import pallas as pl
from jax.experimental.pallas import tpu as pltpu

C = 192
H = 224
W = 224
HEADS = 4
CH = C // HEADS
TR = 8
NT = H // TR
R = TR + 2
N_PIX = H * W
KS = (CH // 2, CH * 2 // 3, CH * 3 // 4, CH * 4 // 5)


def _chain_kernel(xp_ref, x_ref, xn_ref, lw_ref, lb_ref, w_ref, b_ref,
                  w9_ref, bd_ref, o_ref, s_ref):
    i = pl.program_id(0)
    y = jnp.concatenate([xp_ref[:, TR - 1:TR, :], x_ref[...],
                         xn_ref[:, 0:1, :]], axis=1)
    x = y.reshape(C, R * W)
    mu = jnp.mean(x, axis=0, keepdims=True)
    var = jnp.mean(x * x, axis=0, keepdims=True) - mu * mu
    x = (x - mu) * lax.rsqrt(var + 1e-6) * lw_ref[...] + lb_ref[...]
    z = lax.dot_general(w_ref[...].astype(jnp.bfloat16),
                        x.astype(jnp.bfloat16), (((1,), (0,)), ((), ())),
                        preferred_element_type=jnp.float32) + b_ref[...]
    yc = z.reshape(C, R, W)
    row = lax.broadcasted_iota(jnp.int32, (1, R, 1), 1)
    topf = (i > 0).astype(jnp.float32)
    botf = (i < NT - 1).astype(jnp.float32)
    rowmask = jnp.where(row == 0, topf, 1.0) * \
        jnp.where(row == R - 1, botf, 1.0)
    yc = yc * rowmask
    lane = lax.broadcasted_iota(jnp.int32, (1, 1, W), 2)
    zz = None
    for dc in range(3):
        s = None
        for dr in range(3):
            t = w9_ref[:, 3 * dr + dc:3 * dr + dc + 1][:, :, None] \
                * yc[:, dr:dr + TR, :]
            s = t if s is None else s + t
        if dc == 0:
            s = jnp.where(lane > 0, pltpu.roll(s, 1, 2), 0.0)
        elif dc == 2:
            s = jnp.where(lane < W - 1, pltpu.roll(s, W - 1, 2), 0.0)
        zz = s if zz is None else zz + s
    o = zz + bd_ref[...][:, :, None]
    o_ref[...] = o
    of = o.reshape(C, TR * W)

    @pl.when(i == 0)
    def _():
        s_ref[...] = jnp.zeros_like(s_ref)

    s_ref[...] += jnp.sum(of * of, axis=1, keepdims=True)


def _g_kernel(q_ref, k_ref, g_ref):
    i = pl.program_id(0)
    q = q_ref[...].reshape(C, TR * W)
    k = k_ref[...].reshape(C, TR * W)

    @pl.when(i == 0)
    def _():
        g_ref[...] = jnp.zeros_like(g_ref)

    for h in range(HEADS):
        g_ref[h, :, :] += lax.dot_general(
            q[h * CH:(h + 1) * CH, :], k[h * CH:(h + 1) * CH, :],
            (((1,), (1,)), ((), ())), preferred_element_type=jnp.float32)


def _p2_kernel(g_ref, sq_ref, sk_ref, t_ref, aw_ref, wproj_ref, m_ref):
    rq = 1.0 / jnp.maximum(jnp.sqrt(sq_ref[...]), 1e-12)
    rk = 1.0 / jnp.maximum(jnp.sqrt(sk_ref[...]), 1e-12)
    temp = t_ref[0:1, 0:1]
    rows = []
    for h in range(HEADS):
        d = lax.dot_general(rq[h * CH:(h + 1) * CH, :],
                            rk[h * CH:(h + 1) * CH, :],
                            (((1,), (1,)), ((), ())),
                            preferred_element_type=jnp.float32)
        rows.append(g_ref[h, :, :] * d)
    attn = jnp.concatenate(rows, axis=0) * temp

    a1 = attn[:, :, None]
    a2 = attn[:, None, :]
    ei = lax.broadcasted_iota(jnp.int32, (1, CH, CH), 1)
    di = lax.broadcasted_iota(jnp.int32, (1, CH, CH), 2)
    first = (a1 > a2) | ((a1 == a2) & (ei < di))
    rank = jnp.sum(first.astype(jnp.float32), axis=1)

    m = jnp.max(attn, axis=1, keepdims=True)
    e = jnp.exp(attn - m)
    wsum = jnp.zeros_like(attn)
    for idx, kk in enumerate(KS):
        mask = (rank < kk).astype(jnp.float32)
        den = jnp.sum(e * mask, axis=1, keepdims=True)
        wsum += aw_ref[idx:idx + 1, 0:1] * (mask / den)
    a = e * wsum

    for h in range(HEADS):
        mh = lax.dot_general(wproj_ref[:, h * CH:(h + 1) * CH],
                             a[h * CH:(h + 1) * CH, :],
                             (((1,), (0,)), ((), ())),
                             preferred_element_type=jnp.float32)
        m_ref[:, h * CH:(h + 1) * CH] = mh


def _p3_kernel(v_ref, xc_ref, xd_ref, m_ref, bp_ref, w1_ref, w2_ref,
               f1_ref, s_ref, acc_ref):
    i = pl.program_id(0)
    vf = v_ref[...].reshape(C, TR * W)
    f1 = lax.dot_general(m_ref[...].astype(jnp.bfloat16),
                         vf.astype(jnp.bfloat16), (((1,), (0,)), ((), ())),
                         preferred_element_type=jnp.float32) + bp_ref[...]
    f1 = f1.reshape(C, TR, W) + xc_ref[...]
    f1_ref[...] = f1

    @pl.when(i == 0)
    def _():
        acc_ref[...] = jnp.zeros_like(acc_ref)

    acc_ref[...] += (jnp.sum(f1.reshape(C, TR * W), axis=1, keepdims=True)
                     + jnp.sum(xd_ref[...].reshape(C, TR * W), axis=1,
                               keepdims=True))

    @pl.when(i == NT - 1)
    def _():
        pooled = acc_ref[...] * (1.0 / N_PIX)
        hid = jnp.maximum(
            lax.dot_general(w1_ref[...], pooled, (((1,), (0,)), ((), ())),
                            preferred_element_type=jnp.float32), 0.0)
        att = lax.dot_general(w2_ref[...], hid, (((1,), (0,)), ((), ())),
                              preferred_element_type=jnp.float32)
        pair = jnp.concatenate([att[0:C, :], att[C:2 * C, :]], axis=1)
        mx = jnp.max(pair, axis=1, keepdims=True)
        ee = jnp.exp(pair - mx)
        s_ref[...] = ee / jnp.sum(ee, axis=1, keepdims=True)


def _p5_kernel(f1_ref, xd_ref, s_ref, o_ref):
    s0 = s_ref[:, 0:1][:, :, None]
    s1 = s_ref[:, 1:2][:, :, None]
    o_ref[...] = f1_ref[...] * s0 + xd_ref[...] * s1


def kernel(T_E_input, C_E_input, T_D_input, t_norm_w, t_norm_b, c_norm_w,
           c_norm_b, Wq, bq, Wq_dw, bq_dw, Wk, bk, Wk_dw, bk_dw, Wv, bv,
           Wv_dw, bv_dw, temperature, Wproj, bproj, a1, a2, a3, a4,
           sk_w1, sk_w2):
    f32 = jnp.float32
    xt = T_E_input.reshape(C, H, W)
    xc = C_E_input.reshape(C, H, W)
    xd = T_D_input.reshape(C, H, W)
    col = lambda v: v.reshape(-1, 1).astype(f32)
    temp2 = temperature.reshape(1, 1)
    awts = jnp.concatenate([a1, a2, a3, a4]).reshape(4, 1)

    main = pl.BlockSpec((C, TR, W), lambda i: (0, i, 0))
    halo_t = pl.BlockSpec((C, TR, W), lambda i: (0, jnp.maximum(i - 1, 0), 0))
    halo_b = pl.BlockSpec((C, TR, W),
                          lambda i: (0, jnp.minimum(i + 1, NT - 1), 0))

    def const(shape):
        nd = len(shape)
        return pl.BlockSpec(shape, lambda i: (0,) * nd)

    seq = pltpu.CompilerParams(dimension_semantics=("arbitrary",),
                               vmem_limit_bytes=100 * 1024 * 1024)

    chain = pl.pallas_call(
        _chain_kernel,
        grid=(NT,),
        in_specs=[halo_t, main, halo_b,
                  const((C, 1)), const((C, 1)), const((C, C)), const((C, 1)),
                  const((C, 9)), const((C, 1))],
        out_specs=[main, const((C, 1))],
        out_shape=[jax.ShapeDtypeStruct((C, H, W), f32),
                   jax.ShapeDtypeStruct((C, 1), f32)],
        compiler_params=seq,
    )

    q, sq = chain(xt, xt, xt, col(t_norm_w), col(t_norm_b),
                  Wq, col(bq), Wq_dw.reshape(C, 9), col(bq_dw))
    k, sk = chain(xc, xc, xc, col(c_norm_w), col(c_norm_b),
                  Wk, col(bk), Wk_dw.reshape(C, 9), col(bk_dw))
    v, _ = chain(xc, xc, xc, col(c_norm_w), col(c_norm_b),
                 Wv, col(bv), Wv_dw.reshape(C, 9), col(bv_dw))

    g = pl.pallas_call(
        _g_kernel,
        grid=(NT,),
        in_specs=[main, main],
        out_specs=const((HEADS, CH, CH)),
        out_shape=jax.ShapeDtypeStruct((HEADS, CH, CH), f32),
        compiler_params=seq,
    )(q, k)

    m = pl.pallas_call(
        _p2_kernel,
        out_shape=jax.ShapeDtypeStruct((C, C), f32),
    )(g, sq, sk, temp2, awts, Wproj)

    f1, s = pl.pallas_call(
        _p3_kernel,
        grid=(NT,),
        in_specs=[main, main, main, const((C, C)), const((C, 1)),
                  const(sk_w1.shape), const(sk_w2.shape)],
        out_specs=[main, const((C, 2))],
        out_shape=[jax.ShapeDtypeStruct((C, H, W), f32),
                   jax.ShapeDtypeStruct((C, 2), f32)],
        scratch_shapes=[pltpu.VMEM((C, 1), f32)],
        compiler_params=seq,
    )(v, xc, xd, m, col(bproj), sk_w1, sk_w2)

    out = pl.pallas_call(
        _p5_kernel,
        grid=(NT,),
        in_specs=[main, main, const((C, 2))],
        out_specs=main,
        out_shape=jax.ShapeDtypeStruct((C, H, W), f32),
        compiler_params=seq,
    )(f1, xd, s)

    return out.reshape(1, C, H, W)


# speedup vs baseline: 1.1148x; 1.0324x over previous
import jax
import jax.numpy as jnp
from jax import lax
from jax.experimental import pallas as pl
from jax.experimental.pallas import tpu as pltpu

C = 192
H = 224
W = 224
HEADS = 4
CH = C // HEADS
TR = 8
NT = H // TR
R = TR + 2
N_PIX = H * W
KS = (CH // 2, CH * 2 // 3, CH * 3 // 4, CH * 4 // 5)


def _chain_kernel(xp_ref, x_ref, xn_ref, lw_ref, lb_ref, w_ref, b_ref,
                  w9_ref, bd_ref, o_ref, s_ref):
    i = pl.program_id(0)
    y = jnp.concatenate([xp_ref[:, TR - 1:TR, :], x_ref[...],
                         xn_ref[:, 0:1, :]], axis=1)
    x = y.reshape(C, R * W)
    mu = jnp.mean(x, axis=0, keepdims=True)
    var = jnp.mean(x * x, axis=0, keepdims=True) - mu * mu
    x = (x - mu) * lax.rsqrt(var + 1e-6) * lw_ref[...] + lb_ref[...]
    z = lax.dot_general(w_ref[...].astype(jnp.bfloat16),
                        x.astype(jnp.bfloat16), (((1,), (0,)), ((), ())),
                        preferred_element_type=jnp.float32) + b_ref[...]
    yc = z.reshape(C, R, W)
    row = lax.broadcasted_iota(jnp.int32, (1, R, 1), 1)
    topf = (i > 0).astype(jnp.float32)
    botf = (i < NT - 1).astype(jnp.float32)
    rowmask = jnp.where(row == 0, topf, 1.0) * \
        jnp.where(row == R - 1, botf, 1.0)
    yc = yc * rowmask
    lane = lax.broadcasted_iota(jnp.int32, (1, 1, W), 2)
    zz = None
    for dc in range(3):
        s = None
        for dr in range(3):
            t = w9_ref[:, 3 * dr + dc:3 * dr + dc + 1][:, :, None] \
                * yc[:, dr:dr + TR, :]
            s = t if s is None else s + t
        if dc == 0:
            s = jnp.where(lane > 0, pltpu.roll(s, 1, 2), 0.0)
        elif dc == 2:
            s = jnp.where(lane < W - 1, pltpu.roll(s, W - 1, 2), 0.0)
        zz = s if zz is None else zz + s
    o = zz + bd_ref[...][:, :, None]
    o_ref[...] = o
    of = o.reshape(C, TR * W)

    @pl.when(i == 0)
    def _():
        s_ref[...] = jnp.zeros_like(s_ref)

    s_ref[...] += jnp.sum(of * of, axis=1, keepdims=True)


def _k_chain_kernel(xp_ref, x_ref, xn_ref, q_ref, lw_ref, lb_ref, w_ref,
                    b_ref, w9_ref, bd_ref, g_ref, sk_ref):
    i = pl.program_id(0)
    y = jnp.concatenate([xp_ref[:, TR - 1:TR, :], x_ref[...],
                         xn_ref[:, 0:1, :]], axis=1)
    x = y.reshape(C, R * W)
    mu = jnp.mean(x, axis=0, keepdims=True)
    var = jnp.mean(x * x, axis=0, keepdims=True) - mu * mu
    x = (x - mu) * lax.rsqrt(var + 1e-6) * lw_ref[...] + lb_ref[...]
    z = lax.dot_general(w_ref[...].astype(jnp.bfloat16),
                        x.astype(jnp.bfloat16), (((1,), (0,)), ((), ())),
                        preferred_element_type=jnp.float32) + b_ref[...]
    yc = z.reshape(C, R, W)
    row = lax.broadcasted_iota(jnp.int32, (1, R, 1), 1)
    topf = (i > 0).astype(jnp.float32)
    botf = (i < NT - 1).astype(jnp.float32)
    rowmask = jnp.where(row == 0, topf, 1.0) * \
        jnp.where(row == R - 1, botf, 1.0)
    yc = yc * rowmask
    lane = lax.broadcasted_iota(jnp.int32, (1, 1, W), 2)
    zz = None
    for dc in range(3):
        s = None
        for dr in range(3):
            t = w9_ref[:, 3 * dr + dc:3 * dr + dc + 1][:, :, None] \
                * yc[:, dr:dr + TR, :]
            s = t if s is None else s + t
        if dc == 0:
            s = jnp.where(lane > 0, pltpu.roll(s, 1, 2), 0.0)
        elif dc == 2:
            s = jnp.where(lane < W - 1, pltpu.roll(s, W - 1, 2), 0.0)
        zz = s if zz is None else zz + s
    k = (zz + bd_ref[...][:, :, None]).reshape(C, TR * W)
    q = q_ref[...].reshape(C, TR * W)

    @pl.when(i == 0)
    def _():
        g_ref[...] = jnp.zeros_like(g_ref)
        sk_ref[...] = jnp.zeros_like(sk_ref)

    sk_ref[...] += jnp.sum(k * k, axis=1, keepdims=True)
    for h in range(HEADS):
        g_ref[h, :, :] += lax.dot_general(
            q[h * CH:(h + 1) * CH, :], k[h * CH:(h + 1) * CH, :],
            (((1,), (1,)), ((), ())), preferred_element_type=jnp.float32)


def _p2_kernel(g_ref, sq_ref, sk_ref, t_ref, aw_ref, wproj_ref, m_ref):
    rq = 1.0 / jnp.maximum(jnp.sqrt(sq_ref[...]), 1e-12)
    rk = 1.0 / jnp.maximum(jnp.sqrt(sk_ref[...]), 1e-12)
    temp = t_ref[0:1, 0:1]
    rows = []
    for h in range(HEADS):
        d = lax.dot_general(rq[h * CH:(h + 1) * CH, :],
                            rk[h * CH:(h + 1) * CH, :],
                            (((1,), (1,)), ((), ())),
                            preferred_element_type=jnp.float32)
        rows.append(g_ref[h, :, :] * d)
    attn = jnp.concatenate(rows, axis=0) * temp

    a1 = attn[:, :, None]
    a2 = attn[:, None, :]
    ei = lax.broadcasted_iota(jnp.int32, (1, CH, CH), 1)
    di = lax.broadcasted_iota(jnp.int32, (1, CH, CH), 2)
    first = (a1 > a2) | ((a1 == a2) & (ei < di))
    rank = jnp.sum(first.astype(jnp.float32), axis=1)

    m = jnp.max(attn, axis=1, keepdims=True)
    e = jnp.exp(attn - m)
    wsum = jnp.zeros_like(attn)
    for idx, kk in enumerate(KS):
        mask = (rank < kk).astype(jnp.float32)
        den = jnp.sum(e * mask, axis=1, keepdims=True)
        wsum += aw_ref[idx:idx + 1, 0:1] * (mask / den)
    a = e * wsum

    for h in range(HEADS):
        mh = lax.dot_general(wproj_ref[:, h * CH:(h + 1) * CH],
                             a[h * CH:(h + 1) * CH, :],
                             (((1,), (0,)), ((), ())),
                             preferred_element_type=jnp.float32)
        m_ref[:, h * CH:(h + 1) * CH] = mh


def _p3_kernel(v_ref, xc_ref, xd_ref, m_ref, bp_ref, w1_ref, w2_ref,
               f1_ref, s_ref, acc_ref):
    i = pl.program_id(0)
    vf = v_ref[...].reshape(C, TR * W)
    f1 = lax.dot_general(m_ref[...].astype(jnp.bfloat16),
                         vf.astype(jnp.bfloat16), (((1,), (0,)), ((), ())),
                         preferred_element_type=jnp.float32) + bp_ref[...]
    f1 = f1.reshape(C, TR, W) + xc_ref[...]
    f1_ref[...] = f1

    @pl.when(i == 0)
    def _():
        acc_ref[...] = jnp.zeros_like(acc_ref)

    acc_ref[...] += (jnp.sum(f1.reshape(C, TR * W), axis=1, keepdims=True)
                     + jnp.sum(xd_ref[...].reshape(C, TR * W), axis=1,
                               keepdims=True))

    @pl.when(i == NT - 1)
    def _():
        pooled = acc_ref[...] * (1.0 / N_PIX)
        hid = jnp.maximum(
            lax.dot_general(w1_ref[...], pooled, (((1,), (0,)), ((), ())),
                            preferred_element_type=jnp.float32), 0.0)
        att = lax.dot_general(w2_ref[...], hid, (((1,), (0,)), ((), ())),
                              preferred_element_type=jnp.float32)
        pair = jnp.concatenate([att[0:C, :], att[C:2 * C, :]], axis=1)
        mx = jnp.max(pair, axis=1, keepdims=True)
        ee = jnp.exp(pair - mx)
        s_ref[...] = ee / jnp.sum(ee, axis=1, keepdims=True)


def _p5_kernel(f1_ref, xd_ref, s_ref, o_ref):
    s0 = s_ref[:, 0:1][:, :, None]
    s1 = s_ref[:, 1:2][:, :, None]
    o_ref[...] = f1_ref[...] * s0 + xd_ref[...] * s1


def kernel(T_E_input, C_E_input, T_D_input, t_norm_w, t_norm_b, c_norm_w,
           c_norm_b, Wq, bq, Wq_dw, bq_dw, Wk, bk, Wk_dw, bk_dw, Wv, bv,
           Wv_dw, bv_dw, temperature, Wproj, bproj, a1, a2, a3, a4,
           sk_w1, sk_w2):
    f32 = jnp.float32
    xt = T_E_input.reshape(C, H, W)
    xc = C_E_input.reshape(C, H, W)
    xd = T_D_input.reshape(C, H, W)
    col = lambda v: v.reshape(-1, 1).astype(f32)
    temp2 = temperature.reshape(1, 1)
    awts = jnp.concatenate([a1, a2, a3, a4]).reshape(4, 1)

    main = pl.BlockSpec((C, TR, W), lambda i: (0, i, 0))
    halo_t = pl.BlockSpec((C, TR, W), lambda i: (0, jnp.maximum(i - 1, 0), 0))
    halo_b = pl.BlockSpec((C, TR, W),
                          lambda i: (0, jnp.minimum(i + 1, NT - 1), 0))

    def const(shape):
        nd = len(shape)
        return pl.BlockSpec(shape, lambda i: (0,) * nd)

    seq = pltpu.CompilerParams(dimension_semantics=("arbitrary",),
                               vmem_limit_bytes=100 * 1024 * 1024)

    chain = pl.pallas_call(
        _chain_kernel,
        grid=(NT,),
        in_specs=[halo_t, main, halo_b,
                  const((C, 1)), const((C, 1)), const((C, C)), const((C, 1)),
                  const((C, 9)), const((C, 1))],
        out_specs=[main, const((C, 1))],
        out_shape=[jax.ShapeDtypeStruct((C, H, W), f32),
                   jax.ShapeDtypeStruct((C, 1), f32)],
        compiler_params=seq,
    )

    q, sq = chain(xt, xt, xt, col(t_norm_w), col(t_norm_b),
                  Wq, col(bq), Wq_dw.reshape(C, 9), col(bq_dw))
    v, _ = chain(xc, xc, xc, col(c_norm_w), col(c_norm_b),
                 Wv, col(bv), Wv_dw.reshape(C, 9), col(bv_dw))

    g, sk = pl.pallas_call(
        _k_chain_kernel,
        grid=(NT,),
        in_specs=[halo_t, main, halo_b, main,
                  const((C, 1)), const((C, 1)), const((C, C)), const((C, 1)),
                  const((C, 9)), const((C, 1))],
        out_specs=[const((HEADS, CH, CH)), const((C, 1))],
        out_shape=[jax.ShapeDtypeStruct((HEADS, CH, CH), f32),
                   jax.ShapeDtypeStruct((C, 1), f32)],
        compiler_params=seq,
    )(xc, xc, xc, q, col(c_norm_w), col(c_norm_b),
      Wk, col(bk), Wk_dw.reshape(C, 9), col(bk_dw))

    m = pl.pallas_call(
        _p2_kernel,
        out_shape=jax.ShapeDtypeStruct((C, C), f32),
    )(g, sq, sk, temp2, awts, Wproj)

    f1, s = pl.pallas_call(
        _p3_kernel,
        grid=(NT,),
        in_specs=[main, main, main, const((C, C)), const((C, 1)),
                  const(sk_w1.shape), const(sk_w2.shape)],
        out_specs=[main, const((C, 2))],
        out_shape=[jax.ShapeDtypeStruct((C, H, W), f32),
                   jax.ShapeDtypeStruct((C, 2), f32)],
        scratch_shapes=[pltpu.VMEM((C, 1), f32)],
        compiler_params=seq,
    )(v, xc, xd, m, col(bproj), sk_w1, sk_w2)

    out = pl.pallas_call(
        _p5_kernel,
        grid=(NT,),
        in_specs=[main, main, const((C, 2))],
        out_specs=main,
        out_shape=jax.ShapeDtypeStruct((C, H, W), f32),
        compiler_params=seq,
    )(f1, xd, s)

    return out.reshape(1, C, H, W)
